# baseline (speedup 1.0000x reference)
import functools

import jax
import jax.numpy as jnp
from jax import lax
from jax.experimental import pallas as pl
from jax.experimental.pallas import tpu as pltpu
from jax.experimental.pallas import tpu_sc as plsc

_NC = 2
_NS = 16
_NW = _NC * _NS
_CHUNK = 128


@functools.lru_cache(maxsize=None)
def _build(n_chunks, vocab, dim):
    mesh = plsc.VectorSubcoreMesh(core_axis_name="c", subcore_axis_name="s")

    @functools.partial(
        pl.kernel,
        mesh=mesh,
        compiler_params=pltpu.CompilerParams(use_tc_tiling_on_sc=False),
        out_type=jax.ShapeDtypeStruct((_NW, n_chunks, _CHUNK, dim), jnp.float32),
        scratch_types=[
            pltpu.VMEM((n_chunks, _CHUNK), jnp.int32),
            pltpu.VMEM((n_chunks, _CHUNK, dim), jnp.float32),
            pltpu.SemaphoreType.DMA,
        ],
    )
    def _gather(idx_hbm, table_hbm, out_hbm, idx_v, rows_v, sem):
        wid = lax.axis_index("s") * _NC + lax.axis_index("c")
        pltpu.sync_copy(idx_hbm.at[wid], idx_v)
        copies = [
            pltpu.async_copy(table_hbm.at[idx_v.at[j]], rows_v.at[j], sem)
            for j in range(n_chunks)
        ]
        for c in copies:
            c.wait()
        pltpu.sync_copy(rows_v, out_hbm.at[wid])

    return _gather


def kernel(token_id, table):
    b = token_id.shape[0]
    vocab, dim = table.shape
    per_w = -(-b // (_NW * _CHUNK))
    b_pad = _NW * per_w * _CHUNK
    idx = token_id.astype(jnp.int32)
    npad = b_pad - b
    if npad:
        pad = jnp.arange(npad, dtype=jnp.int32) % jnp.int32(vocab)
        idx = jnp.concatenate([idx, pad])
    idx3 = idx.reshape(_NW, per_w, _CHUNK)
    out = _build(per_w, vocab, dim)(idx3, table)
    return out.reshape(b_pad, dim)[:b]
